# Initial kernel scaffold; baseline (speedup 1.0000x reference)
import jax
import jax.numpy as jnp
from jax.experimental import pallas as pl


def kernel(x, pos_table):
    raise NotImplementedError("write your pallas kernel here")


# speedup vs baseline: 1.4642x; 1.4642x over previous
import jax
import jax.numpy as jnp
from jax.experimental import pallas as pl


_MAXLEN = 8192
_EMBED = 128
_BATCH = 4
_TBLK = 1024


def _add_body(x_ref, p_ref, o_ref):
    o_ref[...] = x_ref[...] + p_ref[...]


def kernel(x, pos_table):
    grid = (_BATCH, _MAXLEN // _TBLK)
    return pl.pallas_call(
        _add_body,
        grid=grid,
        in_specs=[
            pl.BlockSpec((1, _TBLK, _EMBED), lambda b, t: (b, t, 0)),
            pl.BlockSpec((_TBLK, _EMBED), lambda b, t: (t, 0)),
        ],
        out_specs=pl.BlockSpec((1, _TBLK, _EMBED), lambda b, t: (b, t, 0)),
        out_shape=jax.ShapeDtypeStruct((_BATCH, _MAXLEN, _EMBED), jnp.float32),
    )(x, pos_table)
